# baseline (speedup 1.0000x reference)
import functools

import jax
import jax.numpy as jnp
from jax import lax
from jax.experimental import pallas as pl
from jax.experimental.pallas import tpu as pltpu
from jax.experimental.pallas import tpu_sc as plsc

VOCAB = 100000
HIDDEN = 64
TOKENS = 256
BLK = 2048


def _sc_gather(table2, ids_half):
    info = plsc.get_sparse_core_info()
    nc, ns = info.num_cores, info.num_subcores
    nw = nc * ns
    b_per_w = TOKENS // nw
    mesh = plsc.VectorSubcoreMesh(core_axis_name="c", subcore_axis_name="s")

    @functools.partial(
        pl.kernel,
        mesh=mesh,
        out_type=jax.ShapeDtypeStruct((TOKENS, 2 * HIDDEN), jnp.float32),
        scratch_types=[
            pltpu.VMEM((b_per_w,), jnp.int32),
            pltpu.VMEM((b_per_w, 2 * HIDDEN), jnp.float32),
            pltpu.SemaphoreType.DMA,
        ],
    )
    def gather_kernel(table_hbm, idx_hbm, out_hbm, idx_v, rows_v, sem):
        wid = lax.axis_index("s") * nc + lax.axis_index("c")
        base = wid * b_per_w
        pltpu.sync_copy(idx_hbm.at[pl.ds(base, b_per_w)], idx_v)
        pltpu.async_copy(table_hbm.at[idx_v], rows_v, sem).wait()
        pltpu.sync_copy(rows_v, out_hbm.at[pl.ds(base, b_per_w)])

    return gather_kernel(table2, ids_half)


def _matmul_body(h2_ref, par_ref, w_ref, out_ref, h_ref):
    @pl.when(pl.program_id(0) == 0)
    def _():
        odd = par_ref[...] == 1
        h_ref[...] = jnp.where(odd, h2_ref[:, HIDDEN:], h2_ref[:, :HIDDEN])

    out_ref[...] = lax.dot_general(
        h_ref[...], w_ref[...],
        dimension_numbers=(((1,), (1,)), ((), ())),
        preferred_element_type=jnp.float32,
    )


def _tc_logits(h2, parity, head_w):
    grid = pl.cdiv(VOCAB, BLK)
    return pl.pallas_call(
        _matmul_body,
        grid=(grid,),
        in_specs=[
            pl.BlockSpec((TOKENS, 2 * HIDDEN), lambda i: (0, 0)),
            pl.BlockSpec((TOKENS, 1), lambda i: (0, 0)),
            pl.BlockSpec((BLK, HIDDEN), lambda i: (i, 0)),
        ],
        out_specs=pl.BlockSpec((TOKENS, BLK), lambda i: (0, i)),
        out_shape=jax.ShapeDtypeStruct((TOKENS, VOCAB), jnp.float32),
        scratch_shapes=[pltpu.VMEM((TOKENS, HIDDEN), jnp.float32)],
    )(h2, parity, head_w)


def kernel(input_ids, embed, head_w):
    b, l = input_ids.shape
    ids_flat = input_ids.reshape(-1).astype(jnp.int32)
    table2 = embed.reshape(VOCAB // 2, 2 * HIDDEN)
    h2 = _sc_gather(table2, ids_flat // 2)
    parity = (ids_flat % 2).reshape(TOKENS, 1)
    logits = _tc_logits(h2, parity, head_w)
    return logits.reshape(b, l, VOCAB)


# speedup vs baseline: 1.0001x; 1.0001x over previous
import functools

import jax
import jax.numpy as jnp
from jax import lax
from jax.experimental import pallas as pl
from jax.experimental.pallas import tpu as pltpu
from jax.experimental.pallas import tpu_sc as plsc

VOCAB = 100000
HIDDEN = 64
TOKENS = 256
SLAB = 8
BLK = 2048


def _sc_gather(table, ids):
    info = plsc.get_sparse_core_info()
    nc, ns = info.num_cores, info.num_subcores
    nw = nc * ns
    b_per_w = TOKENS // nw
    mesh = plsc.VectorSubcoreMesh(core_axis_name="c", subcore_axis_name="s")

    @functools.partial(
        pl.kernel,
        mesh=mesh,
        out_type=jax.ShapeDtypeStruct((TOKENS, HIDDEN), jnp.float32),
        scratch_types=[
            pltpu.VMEM((b_per_w,), jnp.int32),
            pltpu.VMEM((b_per_w, HIDDEN), jnp.float32),
            pltpu.SemaphoreType.DMA,
        ],
        compiler_params=pltpu.CompilerParams(use_tc_tiling_on_sc=False),
    )
    def gather_kernel(table_hbm, idx_hbm, out_hbm, idx_v, rows_v, sem):
        wid = lax.axis_index("s") * nc + lax.axis_index("c")
        base = wid * b_per_w
        pltpu.sync_copy(idx_hbm.at[pl.ds(base, b_per_w)], idx_v)
        pltpu.async_copy(table_hbm.at[idx_v], rows_v, sem).wait()
        pltpu.sync_copy(rows_v, out_hbm.at[pl.ds(base, b_per_w)])

    return gather_kernel(table, ids)


def _matmul_body(h_ref, w_ref, out_ref):
    out_ref[...] = lax.dot_general(
        h_ref[...], w_ref[...],
        dimension_numbers=(((1,), (1,)), ((), ())),
        preferred_element_type=jnp.float32,
    )


def _tc_logits(h, head_w):
    grid = pl.cdiv(VOCAB, BLK)
    return pl.pallas_call(
        _matmul_body,
        grid=(grid,),
        in_specs=[
            pl.BlockSpec((TOKENS, HIDDEN), lambda i: (0, 0)),
            pl.BlockSpec((BLK, HIDDEN), lambda i: (i, 0)),
        ],
        out_specs=pl.BlockSpec((TOKENS, BLK), lambda i: (0, i)),
        out_shape=jax.ShapeDtypeStruct((TOKENS, VOCAB), jnp.float32),
    )(h, head_w)


def kernel(input_ids, embed, head_w):
    b, l = input_ids.shape
    ids_flat = input_ids.reshape(-1).astype(jnp.int32)
    h = _sc_gather(embed, ids_flat)
    logits = _tc_logits(h, head_w)
    return logits.reshape(b, l, VOCAB)


# speedup vs baseline: 1.5644x; 1.5642x over previous
import functools

import jax
import jax.numpy as jnp
from jax import lax
from jax.experimental import pallas as pl
from jax.experimental.pallas import tpu as pltpu
from jax.experimental.pallas import tpu_sc as plsc

VOCAB = 100000
HIDDEN = 64
TOKENS = 256
BLK = 2048
CHUNK = 128


def _sc_gather(table_flat, flat_idx):
    info = plsc.get_sparse_core_info()
    nc, ns = info.num_cores, info.num_subcores
    nw = nc * ns
    n = TOKENS * HIDDEN
    per_w = n // nw
    mesh = plsc.VectorSubcoreMesh(core_axis_name="c", subcore_axis_name="s")

    @functools.partial(
        pl.kernel,
        mesh=mesh,
        out_type=jax.ShapeDtypeStruct((n,), jnp.float32),
        scratch_types=[
            pltpu.VMEM((per_w,), jnp.int32),
            pltpu.VMEM((per_w,), jnp.float32),
            pltpu.SemaphoreType.DMA,
        ],
        compiler_params=pltpu.CompilerParams(use_tc_tiling_on_sc=False),
    )
    def gather_kernel(table_hbm, idx_hbm, out_hbm, idx_v, rows_v, sem):
        wid = lax.axis_index("s") * nc + lax.axis_index("c")
        base = wid * per_w
        pltpu.sync_copy(idx_hbm.at[pl.ds(base, per_w)], idx_v)
        copies = [
            pltpu.async_copy(
                table_hbm.at[idx_v.at[pl.ds(k * CHUNK, CHUNK)]],
                rows_v.at[pl.ds(k * CHUNK, CHUNK)],
                sem,
            )
            for k in range(per_w // CHUNK)
        ]
        for c in copies:
            c.wait()
        pltpu.sync_copy(rows_v, out_hbm.at[pl.ds(base, per_w)])

    return gather_kernel(table_flat, flat_idx)


def _matmul_body(h_ref, w_ref, out_ref):
    out_ref[...] = lax.dot_general(
        h_ref[...], w_ref[...],
        dimension_numbers=(((1,), (0,)), ((), ())),
        preferred_element_type=jnp.float32,
    )


def _tc_logits(h, head_w_t):
    grid = pl.cdiv(VOCAB, BLK)
    return pl.pallas_call(
        _matmul_body,
        grid=(grid,),
        in_specs=[
            pl.BlockSpec((TOKENS, HIDDEN), lambda i: (0, 0)),
            pl.BlockSpec((HIDDEN, BLK), lambda i: (0, i)),
        ],
        out_specs=pl.BlockSpec((TOKENS, BLK), lambda i: (0, i)),
        out_shape=jax.ShapeDtypeStruct((TOKENS, VOCAB), jnp.float32),
    )(h, head_w_t)


def kernel(input_ids, embed, head_w):
    b, l = input_ids.shape
    ids_flat = input_ids.reshape(-1).astype(jnp.int32)
    flat_idx = (ids_flat[:, None] + (jnp.arange(HIDDEN, dtype=jnp.int32) * VOCAB)[None, :]).reshape(-1)
    table_flat = embed.T.reshape(-1)
    h = _sc_gather(table_flat, flat_idx).reshape(TOKENS, HIDDEN)
    logits = _tc_logits(h, head_w.T)
    return logits.reshape(b, l, VOCAB)
